# Initial kernel scaffold; baseline (speedup 1.0000x reference)
import jax
import jax.numpy as jnp
from jax.experimental import pallas as pl


def kernel(hours, hour_table, W1, b1, W2, b2):
    raise NotImplementedError("write your pallas kernel here")


# speedup vs baseline: 1.2165x; 1.2165x over previous
import functools
import math

import jax
import jax.numpy as jnp
from jax import lax
from jax.experimental import pallas as pl
from jax.experimental.pallas import tpu as pltpu
from jax.experimental.pallas import tpu_sc as plsc

_NUM_CORES = 2
_NUM_SUBCORES = 16
_NW = _NUM_CORES * _NUM_SUBCORES
_CHUNK = 128


def _table_body(tab_ref, w1_ref, b1_ref, w2_ref, b2_ref, out_ref):
    nb = tab_ref.shape[0]
    h = w2_ref.shape[0]
    hour = lax.broadcasted_iota(jnp.int32, (nb, h), 0).astype(jnp.float32)
    ang = 2.0 * math.pi * hour / 24.0
    s = jnp.sin(ang)
    c = jnp.cos(ang)
    hidden = jnp.maximum(s * w1_ref[0:1, :] + c * w1_ref[1:2, :] + b1_ref[:], 0.0)
    cont = jnp.dot(hidden, w2_ref[:], preferred_element_type=jnp.float32) + b2_ref[:]
    out_ref[:, : tab_ref.shape[1]] = tab_ref[:]
    out_ref[:, tab_ref.shape[1] :] = cont


def _build_table(hour_table, W1, b1, W2, b2):
    nb, e = hour_table.shape
    h = W2.shape[0]
    return pl.pallas_call(
        _table_body,
        out_shape=jax.ShapeDtypeStruct((nb, e + h), jnp.float32),
    )(hour_table, W1, b1.reshape(1, h), W2, b2.reshape(1, h))


def _make_gather(n, d):
    assert n % (_NW * _CHUNK) == 0
    bpw = n // _NW
    nchunk = bpw // _CHUNK
    mesh = plsc.VectorSubcoreMesh(core_axis_name="c", subcore_axis_name="s")

    @functools.partial(
        pl.kernel,
        mesh=mesh,
        compiler_params=pltpu.CompilerParams(use_tc_tiling_on_sc=False),
        out_type=jax.ShapeDtypeStruct((n, d), jnp.float32),
        scratch_types=[
            pltpu.VMEM((bpw,), jnp.int32),
            pltpu.VMEM((_CHUNK, d), jnp.float32),
            pltpu.SemaphoreType.DMA,
        ],
    )
    def gather_kernel(table_hbm, idx_hbm, out_hbm, idx_v, buf, gsem):
        wid = lax.axis_index("s") * _NUM_CORES + lax.axis_index("c")
        base = wid * bpw
        pltpu.sync_copy(idx_hbm.at[pl.ds(base, bpw)], idx_v)

        def body(ci, _):
            start = pl.multiple_of(ci * _CHUNK, _CHUNK)
            pltpu.async_copy(
                table_hbm.at[idx_v.at[pl.ds(start, _CHUNK)]], buf, gsem
            ).wait()
            pltpu.sync_copy(buf, out_hbm.at[pl.ds(base + start, _CHUNK)])
            return 0

        lax.fori_loop(0, nchunk, body, 0)

    return gather_kernel


def kernel(hours, hour_table, W1, b1, W2, b2):
    table = _build_table(hour_table, W1, b1, W2, b2)
    flat = hours.reshape(-1)
    n = flat.shape[0]
    d = table.shape[1]
    out = _make_gather(n, d)(table, flat)
    return out.reshape(*hours.shape, d)


# speedup vs baseline: 1.2203x; 1.0031x over previous
import functools
import math

import jax
import jax.numpy as jnp
from jax import lax
from jax.experimental import pallas as pl
from jax.experimental.pallas import tpu as pltpu
from jax.experimental.pallas import tpu_sc as plsc

_NUM_CORES = 2
_NUM_SUBCORES = 16
_NW = _NUM_CORES * _NUM_SUBCORES
_CHUNK = 128


def _table_body(tab_ref, w1_ref, b1_ref, w2_ref, b2_ref, out_ref):
    nb = tab_ref.shape[0]
    h = w2_ref.shape[0]
    hour = lax.broadcasted_iota(jnp.int32, (nb, h), 0).astype(jnp.float32)
    ang = 2.0 * math.pi * hour / 24.0
    s = jnp.sin(ang)
    c = jnp.cos(ang)
    hidden = jnp.maximum(s * w1_ref[0:1, :] + c * w1_ref[1:2, :] + b1_ref[:], 0.0)
    cont = jnp.dot(hidden, w2_ref[:], preferred_element_type=jnp.float32) + b2_ref[:]
    out_ref[:, : tab_ref.shape[1]] = tab_ref[:]
    out_ref[:, tab_ref.shape[1] :] = cont


def _build_table(hour_table, W1, b1, W2, b2):
    nb, e = hour_table.shape
    h = W2.shape[0]
    return pl.pallas_call(
        _table_body,
        out_shape=jax.ShapeDtypeStruct((nb, e + h), jnp.float32),
    )(hour_table, W1, b1.reshape(1, h), W2, b2.reshape(1, h))


def _make_gather(n, d):
    assert n % (_NW * _CHUNK) == 0
    bpw = n // _NW
    nchunk = bpw // _CHUNK
    mesh = plsc.VectorSubcoreMesh(core_axis_name="c", subcore_axis_name="s")

    assert nchunk % 2 == 0

    @functools.partial(
        pl.kernel,
        mesh=mesh,
        compiler_params=pltpu.CompilerParams(use_tc_tiling_on_sc=False),
        out_type=jax.ShapeDtypeStruct((n, d), jnp.float32),
        scratch_types=[
            pltpu.VMEM((bpw,), jnp.int32),
            pltpu.VMEM((_CHUNK, d), jnp.float32),
            pltpu.VMEM((_CHUNK, d), jnp.float32),
            pltpu.SemaphoreType.DMA,
            pltpu.SemaphoreType.DMA,
            pltpu.SemaphoreType.DMA,
            pltpu.SemaphoreType.DMA,
        ],
    )
    def gather_kernel(table_hbm, idx_hbm, out_hbm, idx_v, buf0, buf1, g0, g1, w0, w1):
        wid = lax.axis_index("s") * _NUM_CORES + lax.axis_index("c")
        base = wid * bpw
        pltpu.sync_copy(idx_hbm.at[pl.ds(base, bpw)], idx_v)
        bufs = (buf0, buf1)
        gsems = (g0, g1)
        wsems = (w0, w1)

        def gather_start(c, b):
            start = pl.multiple_of(c * _CHUNK, _CHUNK)
            pltpu.async_copy(
                table_hbm.at[idx_v.at[pl.ds(start, _CHUNK)]], bufs[b], gsems[b]
            )

        def gather_wait(b):
            pltpu.make_async_copy(
                out_hbm.at[pl.ds(0, _CHUNK)], bufs[b], gsems[b]
            ).wait()

        def wb_start(c, b):
            start = pl.multiple_of(c * _CHUNK, _CHUNK)
            pltpu.async_copy(bufs[b], out_hbm.at[pl.ds(base + start, _CHUNK)], wsems[b])

        def wb_wait(b):
            pltpu.make_async_copy(
                bufs[b], out_hbm.at[pl.ds(base, _CHUNK)], wsems[b]
            ).wait()

        for b in range(2):
            gather_start(b, b)

        def body(p, _):
            for b in range(2):
                c = 2 * p + b
                gather_wait(b)
                wb_start(c, b)
                wb_wait(b)
                gather_start(c + 2, b)
            return 0

        lax.fori_loop(0, nchunk // 2 - 1, body, 0)

        for b in range(2):
            gather_wait(b)
            wb_start(nchunk - 2 + b, b)
        for b in range(2):
            wb_wait(b)

    return gather_kernel


def kernel(hours, hour_table, W1, b1, W2, b2):
    table = _build_table(hour_table, W1, b1, W2, b2)
    flat = hours.reshape(-1)
    n = flat.shape[0]
    d = table.shape[1]
    out = _make_gather(n, d)(table, flat)
    return out.reshape(*hours.shape, d)
